# Initial kernel scaffold; baseline (speedup 1.0000x reference)
import jax
import jax.numpy as jnp
from jax.experimental import pallas as pl


def kernel(x, threshold_input, embedding_weight, bias):
    raise NotImplementedError("write your pallas kernel here")


# speedup vs baseline: 9.8676x; 9.8676x over previous
import functools

import jax
import jax.numpy as jnp
from jax import lax
from jax.experimental import pallas as pl
from jax.experimental.pallas import tpu as pltpu
from jax.experimental.pallas import tpu_sc as plsc

NUM_VARS = 1000
NUM_FEATURES = 4
NUM_CATEGS = 100
HIDDEN_DIM = 32
BATCH = 128

NQ = 4
NVB = 8
BQ = BATCH // NQ
VB = 128
TPAD = 128


def _tec_body(x_hbm, sthr_hbm, lut_hbm, table_hbm, bias_hbm, out_hbm,
              xbin_hbm, sthr_v, lut_v, x_v, xbin_v, bias_v,
              idx0_0, idx0_1, idx0_2, idx0_3, idx1_0, idx1_1, idx1_2, idx1_3,
              rows0, rows1, sem_g0, sem_g1, sem_o0, sem_o1):
    idx_bufs = ((idx0_0, idx0_1, idx0_2, idx0_3),
                (idx1_0, idx1_1, idx1_2, idx1_3))
    rows = (rows0, rows1)
    sem_g = (sem_g0, sem_g1)
    sem_o = (sem_o0, sem_o1)

    wid = lax.axis_index("c") * 16 + lax.axis_index("s")
    q = wid // NVB
    vb = wid % NVB
    b0 = q * BQ
    v0 = jnp.minimum(vb * VB, NUM_VARS - VB)

    pltpu.sync_copy(sthr_hbm, sthr_v)
    pltpu.sync_copy(lut_hbm, lut_v)
    pltpu.sync_copy(x_hbm.at[pl.ds(b0, BQ), pl.ds(v0, VB)], x_v)
    pltpu.sync_copy(bias_hbm.at[:, pl.ds(v0, VB), :], bias_v)

    iota = lax.iota(jnp.int32, 16)

    def idx_compute(b, par):
        for k in range(8):
            xv = x_v[b, pl.ds(16 * k, 16)]
            r = jnp.zeros((16,), jnp.int32)
            for s in (64, 32, 16, 8, 4, 2, 1):
                vals = plsc.load_gather(sthr_v, [r + (s - 1)])
                r = jnp.where(vals <= xv, r + s, r)
            j = jnp.maximum(r - 1, 0)
            p = plsc.load_gather(lut_v, [j])
            p = jnp.where(r == 0, 0, p)
            p = jnp.where(p == NUM_CATEGS - 1, NUM_CATEGS - 2, p)
            xbin_v[b, pl.ds(16 * k, 16)] = p
            base = (v0 + 16 * k + iota) * (NUM_FEATURES * NUM_CATEGS) + p
            for f in range(4):
                idx_bufs[par][f][pl.ds(16 * k, 16)] = base + f * NUM_CATEGS

    def fire_gathers(b, par):
        for f in range(4):
            pltpu.async_copy(table_hbm.at[idx_bufs[par][f]],
                             rows[par].at[f], sem_g[par])

    def wait_gathers(par):
        for f in range(4):
            pltpu.make_async_copy(table_hbm.at[idx_bufs[par][f]],
                                  rows[par].at[f], sem_g[par]).wait()

    def add_bias(par):
        rbuf = rows[par]
        for f in range(4):
            @pl.loop(0, VB, step=4)
            def _(r0):
                for dr in range(4):
                    for h in range(2):
                        sl = pl.ds(16 * h, 16)
                        plsc.addupdate(rbuf.at[f, r0 + dr, sl],
                                       bias_v[f, r0 + dr, sl])

    def fire_out(b, par):
        for f in range(4):
            pltpu.async_copy(rows[par].at[f],
                             out_hbm.at[b0 + b, pl.ds(v0, VB),
                                        pl.ds(32 * f, 32)], sem_o[par])

    def wait_out(par):
        for f in range(4):
            pltpu.make_async_copy(rows[par].at[f],
                                  out_hbm.at[b0, pl.ds(v0, VB),
                                             pl.ds(32 * f, 32)],
                                  sem_o[par]).wait()

    def finish(b, par):
        wait_gathers(par)
        add_bias(par)
        fire_out(b, par)

    idx_compute(0, 0)
    fire_gathers(0, 0)
    idx_compute(1, 1)
    fire_gathers(1, 1)

    @pl.loop(2, BQ, step=2)
    def _(i):
        finish(i - 2, 0)
        finish(i - 1, 1)
        idx_compute(i, 0)
        wait_out(0)
        fire_gathers(i, 0)
        idx_compute(i + 1, 1)
        wait_out(1)
        fire_gathers(i + 1, 1)

    finish(BQ - 2, 0)
    finish(BQ - 1, 1)
    wait_out(0)
    wait_out(1)

    pltpu.sync_copy(xbin_v, xbin_hbm.at[pl.ds(b0, BQ), pl.ds(v0, VB)])


@jax.jit
def kernel(x, threshold_input, embedding_weight, bias):
    T = NUM_CATEGS
    order = jnp.argsort(threshold_input, stable=True).astype(jnp.int32)
    sthr = threshold_input[order]
    seg_start = jnp.concatenate(
        [jnp.ones((1,), bool), sthr[1:] != sthr[:-1]])
    first = lax.cummax(jnp.where(seg_start, jnp.arange(T, dtype=jnp.int32), 0))
    lut = order[first]
    sthr_p = jnp.concatenate([sthr, jnp.full((TPAD - T,), 2.0, jnp.float32)])
    lut_p = jnp.concatenate([lut, jnp.zeros((TPAD - T,), jnp.int32)])

    x2 = x.reshape(BATCH, NUM_VARS)
    bias_t = bias.reshape(NUM_VARS, NUM_FEATURES, HIDDEN_DIM).transpose(1, 0, 2)

    mesh = plsc.VectorSubcoreMesh(core_axis_name="c", subcore_axis_name="s")
    run = functools.partial(
        pl.kernel,
        out_type=(
            jax.ShapeDtypeStruct((BATCH, NUM_VARS,
                                  NUM_FEATURES * HIDDEN_DIM), jnp.float32),
            jax.ShapeDtypeStruct((BATCH, NUM_VARS), jnp.int32),
        ),
        mesh=mesh,
        compiler_params=pltpu.CompilerParams(use_tc_tiling_on_sc=False,
                                              needs_layout_passes=False),
        scratch_types=[
            pltpu.VMEM((TPAD,), jnp.float32),
            pltpu.VMEM((TPAD,), jnp.int32),
            pltpu.VMEM((BQ, VB), jnp.float32),
            pltpu.VMEM((BQ, VB), jnp.int32),
            pltpu.VMEM((NUM_FEATURES, VB, HIDDEN_DIM), jnp.float32),
            pltpu.VMEM((VB,), jnp.int32),
            pltpu.VMEM((VB,), jnp.int32),
            pltpu.VMEM((VB,), jnp.int32),
            pltpu.VMEM((VB,), jnp.int32),
            pltpu.VMEM((VB,), jnp.int32),
            pltpu.VMEM((VB,), jnp.int32),
            pltpu.VMEM((VB,), jnp.int32),
            pltpu.VMEM((VB,), jnp.int32),
            pltpu.VMEM((NUM_FEATURES, VB, HIDDEN_DIM), jnp.float32),
            pltpu.VMEM((NUM_FEATURES, VB, HIDDEN_DIM), jnp.float32),
            pltpu.SemaphoreType.DMA,
            pltpu.SemaphoreType.DMA,
            pltpu.SemaphoreType.DMA,
            pltpu.SemaphoreType.DMA,
        ],
    )(_tec_body)

    out3, xbin2 = run(x2, sthr_p, lut_p, embedding_weight, bias_t)
    return (out3.reshape(BATCH * NUM_VARS, NUM_FEATURES * HIDDEN_DIM),
            xbin2.reshape(BATCH * NUM_VARS, 1))
